# baseline (speedup 1.0000x reference)
import functools

import jax
import jax.numpy as jnp
from jax import lax
from jax.experimental import pallas as pl
from jax.experimental.pallas import tpu as pltpu
from jax.experimental.pallas import tpu_sc as plsc

VOCAB = 1000000
HIDDEN = 64
EPS = 1e-12

NC = 2
NS = 16
NW = NC * NS

CH = 128
K = 4
ROWS_STEP = CH * K
B_TOTAL = 4096 * 200
B_PER_W = B_TOTAL // NW
N_STEPS = B_PER_W // ROWS_STEP

_mesh = plsc.VectorSubcoreMesh(core_axis_name="c", subcore_axis_name="s")


@functools.partial(
    pl.kernel,
    mesh=_mesh,
    compiler_params=pltpu.CompilerParams(use_tc_tiling_on_sc=False),
    out_type=jax.ShapeDtypeStruct((B_TOTAL, HIDDEN), jnp.float32),
    scratch_types=[
        pltpu.VMEM((K, CH), jnp.int32),
        pltpu.VMEM((K, CH), jnp.int32),
        pltpu.VMEM((ROWS_STEP, 2 * HIDDEN), jnp.float32),
        pltpu.VMEM((ROWS_STEP, HIDDEN), jnp.float32),
        pltpu.SemaphoreType.DMA,
    ],
)
def _sc_gather(idx_hbm, tp_hbm, out_hbm, idx_v, pidx_v, rows2_v, rows_v, sem):
    wid = lax.axis_index("s") * NC + lax.axis_index("c")
    row0 = wid * (B_PER_W // CH)
    iota = lax.iota(jnp.int32, 16)

    def body(j, _):
        r = row0 + j * K
        pltpu.sync_copy(idx_hbm.at[pl.ds(r, K)], idx_v)
        for k in range(K):
            for i in range(CH // 16):
                v = idx_v[k, pl.ds(16 * i, 16)]
                pidx_v[k, pl.ds(16 * i, 16)] = v >> 1
        cps = [
            pltpu.async_copy(
                tp_hbm.at[pidx_v.at[k]],
                rows2_v.at[pl.ds(k * CH, CH)],
                sem,
            )
            for k in range(K)
        ]
        for c in cps:
            c.wait()

        for k in range(K):
            def group(i, _, k=k):
                parf = (idx_v[k, pl.ds(16 * i, 16)] & 1).astype(jnp.float32)
                for l in range(16):
                    rr = k * CH + i * 16 + l
                    hf = jnp.take(parf, jnp.full((16,), l, jnp.int32))
                    for k2 in range(4):
                        lo = rows2_v[rr, pl.ds(16 * k2, 16)]
                        hi = rows2_v[rr, pl.ds(HIDDEN + 16 * k2, 16)]
                        rows_v[rr, pl.ds(16 * k2, 16)] = lo + (hi - lo) * hf
                return 0

            lax.fori_loop(0, CH // 16, group, 0)
        pltpu.sync_copy(rows_v, out_hbm.at[pl.ds(r * CH, ROWS_STEP)])
        return 0

    lax.fori_loop(0, N_STEPS, body, 0)


TP_BLOCKS = 3907
TP_ROWS = TP_BLOCKS * 128


def _pack_body(x_ref, o_ref):
    x = x_ref[...]
    r2 = lax.broadcasted_iota(jnp.int32, (128, 256), 0)
    b2 = lax.broadcasted_iota(jnp.int32, (128, 256), 1)
    se = (b2 == 2 * r2).astype(jnp.float32)
    so = (b2 == 2 * r2 + 1).astype(jnp.float32)
    dn = (((1,), (1,)), ((), ()))
    et = lax.dot_general(se, x, dn, preferred_element_type=jnp.float32)
    ot = lax.dot_general(so, x, dn, preferred_element_type=jnp.float32)
    o_ref[...] = jnp.concatenate([et, ot], axis=1)


_pack = pl.pallas_call(
    _pack_body,
    grid=(TP_BLOCKS,),
    in_specs=[pl.BlockSpec((HIDDEN, 256), lambda b: (0, b))],
    out_specs=pl.BlockSpec((128, 128), lambda b: (b, 0)),
    out_shape=jax.ShapeDtypeStruct((TP_ROWS, 128), jnp.float32),
)


_BB = 2048


def _ln_t_body(x_ref, g_ref, b_ref, o_ref):
    x = x_ref[...]
    m = jnp.mean(x, axis=-1, keepdims=True)
    xc = x - m
    v = jnp.mean(xc * xc, axis=-1, keepdims=True)
    y = xc * lax.rsqrt(v + EPS) * g_ref[...] + b_ref[...]
    eye = (
        lax.broadcasted_iota(jnp.int32, (HIDDEN, HIDDEN), 0)
        == lax.broadcasted_iota(jnp.int32, (HIDDEN, HIDDEN), 1)
    ).astype(jnp.float32)
    yt = lax.dot_general(
        eye, y, (((1,), (1,)), ((), ())),
        preferred_element_type=jnp.float32,
    )
    o_ref[...] = yt[None]


_ln_t = pl.pallas_call(
    _ln_t_body,
    grid=(200, 4096 // _BB),
    in_specs=[
        pl.BlockSpec((_BB, HIDDEN), lambda s, b: (s * (4096 // _BB) + b, 0)),
        pl.BlockSpec((HIDDEN,), lambda s, b: (0,)),
        pl.BlockSpec((HIDDEN,), lambda s, b: (0,)),
    ],
    out_specs=pl.BlockSpec((1, HIDDEN, _BB), lambda s, b: (s, 0, b)),
    out_shape=jax.ShapeDtypeStruct((200, HIDDEN, 4096), jnp.float32),
)


def kernel(input_ids, table, gamma, beta):
    B, S = input_ids.shape
    idx_t = input_ids.T.reshape(-1).astype(jnp.int32)
    idx2d = idx_t.reshape(B_TOTAL // CH, CH)
    tp = _pack(table.T)
    rows = _sc_gather(idx2d, tp)
    out3 = _ln_t(rows, gamma, beta)
    return out3.transpose(2, 0, 1)


# speedup vs baseline: 2.2872x; 2.2872x over previous
import functools

import jax
import jax.numpy as jnp
from jax import lax
from jax.experimental import pallas as pl
from jax.experimental.pallas import tpu as pltpu
from jax.experimental.pallas import tpu_sc as plsc

VOCAB = 1000000
HIDDEN = 64
EPS = 1e-12

NC = 2
NS = 16
NW = NC * NS

CH = 128
K = 2
ROWS_STEP = CH * K
B_TOTAL = 4096 * 200
B_PER_W = B_TOTAL // NW
N_STEPS = B_PER_W // ROWS_STEP

_mesh = plsc.VectorSubcoreMesh(core_axis_name="c", subcore_axis_name="s")


@functools.partial(
    pl.kernel,
    mesh=_mesh,
    compiler_params=pltpu.CompilerParams(use_tc_tiling_on_sc=False),
    out_type=jax.ShapeDtypeStruct((B_TOTAL, HIDDEN), jnp.float32),
    scratch_types=[
        pltpu.VMEM((K, CH), jnp.int32),
        pltpu.VMEM((K, CH), jnp.int32),
        pltpu.VMEM((K, CH), jnp.int32),
        pltpu.VMEM((K, CH), jnp.int32),
        pltpu.VMEM((ROWS_STEP, 2 * HIDDEN), jnp.float32),
        pltpu.VMEM((ROWS_STEP, 2 * HIDDEN), jnp.float32),
        pltpu.VMEM((ROWS_STEP, HIDDEN), jnp.float32),
        pltpu.VMEM((ROWS_STEP, HIDDEN), jnp.float32),
        pltpu.SemaphoreType.DMA,
        pltpu.SemaphoreType.DMA,
        pltpu.SemaphoreType.DMA,
        pltpu.SemaphoreType.DMA,
    ],
)
def _sc_gather(idx_hbm, tp_hbm, out_hbm,
               idx0, idx1, pidx0, pidx1, rows2_0, rows2_1, rv0, rv1,
               semg0, semg1, semo0, semo1):
    wid = lax.axis_index("s") * NC + lax.axis_index("c")
    row0 = wid * (B_PER_W // CH)
    iota = lax.iota(jnp.int32, 16)
    bufs = ((idx0, pidx0, rows2_0, rv0, semg0, semo0),
            (idx1, pidx1, rows2_1, rv1, semg1, semo1))

    def issue(j, p):
        idx_v, pidx_v, rows2_v, _, semg, _ = bufs[p]
        r = row0 + j * K
        pltpu.sync_copy(idx_hbm.at[pl.ds(r, K)], idx_v)
        for k in range(K):
            for i in range(CH // 16):
                v = idx_v[k, pl.ds(16 * i, 16)]
                pidx_v[k, pl.ds(16 * i, 16)] = v >> 1
        for k in range(K):
            pltpu.async_copy(
                tp_hbm.at[pidx_v.at[k]],
                rows2_v.at[pl.ds(k * CH, CH)],
                semg,
            )

    def wait_gathers(p):
        _, pidx_v, rows2_v, _, semg, _ = bufs[p]
        for k in range(K):
            pltpu.make_async_copy(
                tp_hbm.at[pidx_v.at[k]],
                rows2_v.at[pl.ds(k * CH, CH)],
                semg,
            ).wait()

    def wait_out(j, p):
        _, _, _, rows_v, _, semo = bufs[p]
        r = row0 + j * K
        pltpu.make_async_copy(
            rows_v, out_hbm.at[pl.ds(r * CH, ROWS_STEP)], semo
        ).wait()

    def select_and_out(j, p):
        idx_v, _, rows2_v, rows_v, _, semo = bufs[p]
        r = row0 + j * K
        for k in range(K):
            def group(i, _, k=k):
                parf = (idx_v[k, pl.ds(16 * i, 16)] & 1).astype(jnp.float32)
                for l in range(16):
                    rr = k * CH + i * 16 + l
                    hf = jnp.take(parf, jnp.full((16,), l, jnp.int32))
                    for k2 in range(4):
                        lo = rows2_v[rr, pl.ds(16 * k2, 16)]
                        hi = rows2_v[rr, pl.ds(HIDDEN + 16 * k2, 16)]
                        rows_v[rr, pl.ds(16 * k2, 16)] = lo + (hi - lo) * hf
                return 0

            lax.fori_loop(0, CH // 16, group, 0)
        pltpu.async_copy(rows_v, out_hbm.at[pl.ds(r * CH, ROWS_STEP)], semo)

    issue(0, 0)

    def body(t, _):
        j0 = 2 * t
        issue(j0 + 1, 1)
        wait_gathers(0)

        @pl.when(t >= 1)
        def _():
            wait_out(j0, 0)

        select_and_out(j0, 0)

        @pl.when(t + 1 < N_STEPS // 2)
        def _():
            issue(j0 + 2, 0)

        wait_gathers(1)

        @pl.when(t >= 1)
        def _():
            wait_out(j0 + 1, 1)

        select_and_out(j0 + 1, 1)
        return 0

    lax.fori_loop(0, N_STEPS // 2, body, 0)
    wait_out(0, 0)
    wait_out(0, 1)


PACK_W = 4096
PACK_BLOCKS = -(-VOCAB // PACK_W)
TP_ROWS = PACK_BLOCKS * (PACK_W // 2)


def _pack_body(x_ref, o_ref):
    x = x_ref[...]
    r2 = lax.broadcasted_iota(jnp.int32, (128, 256), 0)
    b2 = lax.broadcasted_iota(jnp.int32, (128, 256), 1)
    se = (b2 == 2 * r2).astype(jnp.float32)
    so = (b2 == 2 * r2 + 1).astype(jnp.float32)
    dn = (((1,), (1,)), ((), ()))
    for k in range(PACK_W // 256):
        xk = x[:, 256 * k:256 * (k + 1)]
        et = lax.dot_general(se, xk, dn, preferred_element_type=jnp.float32)
        ot = lax.dot_general(so, xk, dn, preferred_element_type=jnp.float32)
        o_ref[pl.ds(128 * k, 128), :] = jnp.concatenate([et, ot], axis=1)


_pack = pl.pallas_call(
    _pack_body,
    grid=(PACK_BLOCKS,),
    in_specs=[pl.BlockSpec((HIDDEN, PACK_W), lambda b: (0, b))],
    out_specs=pl.BlockSpec((PACK_W // 2, 128), lambda b: (b, 0)),
    out_shape=jax.ShapeDtypeStruct((TP_ROWS, 128), jnp.float32),
)


_BB = 4096


def _ln_t_body(x_ref, g_ref, b_ref, o_ref):
    x = x_ref[...]
    m = jnp.mean(x, axis=-1, keepdims=True)
    xc = x - m
    v = jnp.mean(xc * xc, axis=-1, keepdims=True)
    y = xc * lax.rsqrt(v + EPS) * g_ref[...] + b_ref[...]
    eye = (
        lax.broadcasted_iota(jnp.int32, (HIDDEN, HIDDEN), 0)
        == lax.broadcasted_iota(jnp.int32, (HIDDEN, HIDDEN), 1)
    ).astype(jnp.float32)
    yt = lax.dot_general(
        eye, y, (((1,), (1,)), ((), ())),
        preferred_element_type=jnp.float32,
    )
    o_ref[...] = yt[None]


_ln_t = pl.pallas_call(
    _ln_t_body,
    grid=(200, 4096 // _BB),
    in_specs=[
        pl.BlockSpec((_BB, HIDDEN), lambda s, b: (s * (4096 // _BB) + b, 0)),
        pl.BlockSpec((HIDDEN,), lambda s, b: (0,)),
        pl.BlockSpec((HIDDEN,), lambda s, b: (0,)),
    ],
    out_specs=pl.BlockSpec((1, HIDDEN, _BB), lambda s, b: (s, 0, b)),
    out_shape=jax.ShapeDtypeStruct((200, HIDDEN, 4096), jnp.float32),
)


def kernel(input_ids, table, gamma, beta):
    B, S = input_ids.shape
    idx_t = input_ids.T.reshape(-1).astype(jnp.int32)
    idx2d = idx_t.reshape(B_TOTAL // CH, CH)
    tp = _pack(table.T)
    rows = _sc_gather(idx2d, tp)
    out3 = _ln_t(rows, gamma, beta)
    return out3.transpose(2, 0, 1)


# speedup vs baseline: 2.6854x; 1.1741x over previous
import functools

import jax
import jax.numpy as jnp
from jax import lax
from jax.experimental import pallas as pl
from jax.experimental.pallas import tpu as pltpu
from jax.experimental.pallas import tpu_sc as plsc

VOCAB = 1000000
HIDDEN = 64
EPS = 1e-12

NC = 2
NS = 16
NW = NC * NS

CH = 128
K = 2
ROWS_STEP = CH * K
B_TOTAL = 4096 * 200
CHUNKS = 2
B_CHUNK = B_TOTAL // CHUNKS
S_CHUNK = 200 // CHUNKS
B_PER_W = B_CHUNK // NW
N_STEPS = B_PER_W // ROWS_STEP

_mesh = plsc.VectorSubcoreMesh(core_axis_name="c", subcore_axis_name="s")


@functools.partial(
    pl.kernel,
    mesh=_mesh,
    compiler_params=pltpu.CompilerParams(use_tc_tiling_on_sc=False),
    out_type=jax.ShapeDtypeStruct((B_CHUNK, HIDDEN), jnp.float32),
    scratch_types=[
        pltpu.VMEM((K, CH), jnp.int32),
        pltpu.VMEM((K, CH), jnp.int32),
        pltpu.VMEM((K, CH), jnp.int32),
        pltpu.VMEM((K, CH), jnp.int32),
        pltpu.VMEM((ROWS_STEP, 2 * HIDDEN), jnp.float32),
        pltpu.VMEM((ROWS_STEP, 2 * HIDDEN), jnp.float32),
        pltpu.VMEM((ROWS_STEP, HIDDEN), jnp.float32),
        pltpu.VMEM((ROWS_STEP, HIDDEN), jnp.float32),
        pltpu.SemaphoreType.DMA,
        pltpu.SemaphoreType.DMA,
        pltpu.SemaphoreType.DMA,
        pltpu.SemaphoreType.DMA,
    ],
)
def _sc_gather(idx_hbm, tp_hbm, out_hbm,
               idx0, idx1, pidx0, pidx1, rows2_0, rows2_1, rv0, rv1,
               semg0, semg1, semo0, semo1):
    wid = lax.axis_index("s") * NC + lax.axis_index("c")
    row0 = wid * (B_PER_W // CH)
    iota = lax.iota(jnp.int32, 16)
    bufs = ((idx0, pidx0, rows2_0, rv0, semg0, semo0),
            (idx1, pidx1, rows2_1, rv1, semg1, semo1))

    def issue(j, p):
        idx_v, pidx_v, rows2_v, _, semg, _ = bufs[p]
        r = row0 + j * K
        pltpu.sync_copy(idx_hbm.at[pl.ds(r, K)], idx_v)
        for k in range(K):
            for i in range(CH // 16):
                v = idx_v[k, pl.ds(16 * i, 16)]
                pidx_v[k, pl.ds(16 * i, 16)] = v >> 1
        for k in range(K):
            pltpu.async_copy(
                tp_hbm.at[pidx_v.at[k]],
                rows2_v.at[pl.ds(k * CH, CH)],
                semg,
            )

    def wait_gathers(p):
        _, pidx_v, rows2_v, _, semg, _ = bufs[p]
        for k in range(K):
            pltpu.make_async_copy(
                tp_hbm.at[pidx_v.at[k]],
                rows2_v.at[pl.ds(k * CH, CH)],
                semg,
            ).wait()

    def wait_out(j, p):
        _, _, _, rows_v, _, semo = bufs[p]
        r = row0 + j * K
        pltpu.make_async_copy(
            rows_v, out_hbm.at[pl.ds(r * CH, ROWS_STEP)], semo
        ).wait()

    def select_and_out(j, p):
        idx_v, _, rows2_v, rows_v, _, semo = bufs[p]
        r = row0 + j * K
        for k in range(K):
            def group(i, _, k=k):
                parf = (idx_v[k, pl.ds(16 * i, 16)] & 1).astype(jnp.float32)
                for l in range(16):
                    rr = k * CH + i * 16 + l
                    hf = jnp.take(parf, jnp.full((16,), l, jnp.int32))
                    for k2 in range(4):
                        lo = rows2_v[rr, pl.ds(16 * k2, 16)]
                        hi = rows2_v[rr, pl.ds(HIDDEN + 16 * k2, 16)]
                        rows_v[rr, pl.ds(16 * k2, 16)] = lo + (hi - lo) * hf
                return 0

            lax.fori_loop(0, CH // 16, group, 0)
        pltpu.async_copy(rows_v, out_hbm.at[pl.ds(r * CH, ROWS_STEP)], semo)

    issue(0, 0)

    def body(t, _):
        j0 = 2 * t
        issue(j0 + 1, 1)
        wait_gathers(0)

        @pl.when(t >= 1)
        def _():
            wait_out(j0, 0)

        select_and_out(j0, 0)

        @pl.when(t + 1 < N_STEPS // 2)
        def _():
            issue(j0 + 2, 0)

        wait_gathers(1)

        @pl.when(t >= 1)
        def _():
            wait_out(j0 + 1, 1)

        select_and_out(j0 + 1, 1)
        return 0

    lax.fori_loop(0, N_STEPS // 2, body, 0)
    wait_out(0, 0)
    wait_out(0, 1)


PACK_W = 4096
PACK_BLOCKS = -(-VOCAB // PACK_W)
TP_ROWS = PACK_BLOCKS * (PACK_W // 2)


def _pack_body(x_ref, o_ref):
    x = x_ref[...]
    r2 = lax.broadcasted_iota(jnp.int32, (128, 256), 0)
    b2 = lax.broadcasted_iota(jnp.int32, (128, 256), 1)
    se = (b2 == 2 * r2).astype(jnp.float32)
    so = (b2 == 2 * r2 + 1).astype(jnp.float32)
    dn = (((1,), (1,)), ((), ()))
    for k in range(PACK_W // 256):
        xk = x[:, 256 * k:256 * (k + 1)]
        et = lax.dot_general(se, xk, dn, preferred_element_type=jnp.float32)
        ot = lax.dot_general(so, xk, dn, preferred_element_type=jnp.float32)
        o_ref[pl.ds(128 * k, 128), :] = jnp.concatenate([et, ot], axis=1)


_pack = pl.pallas_call(
    _pack_body,
    grid=(PACK_BLOCKS,),
    in_specs=[pl.BlockSpec((HIDDEN, PACK_W), lambda b: (0, b))],
    out_specs=pl.BlockSpec((PACK_W // 2, 128), lambda b: (b, 0)),
    out_shape=jax.ShapeDtypeStruct((TP_ROWS, 128), jnp.float32),
)


_BB = 2048


def _ln_t_body(x_ref, g_ref, b_ref, o_ref):
    x = x_ref[...]
    dn = (((1,), (0,)), ((), ()))
    dnt = (((1,), (1,)), ((), ()))
    jj = lax.broadcasted_iota(jnp.int32, (128, 2), 0)
    hh = lax.broadcasted_iota(jnp.int32, (128, 2), 1)
    m2 = ((jj // HIDDEN) == hh).astype(jnp.float32)
    s = lax.dot_general(x, m2, dn, preferred_element_type=jnp.float32)
    sq = lax.dot_general(x * x, m2, dn, preferred_element_type=jnp.float32)
    mean = s * (1.0 / HIDDEN)
    var = sq * (1.0 / HIDDEN) - mean * mean
    inv = lax.rsqrt(var + EPS)
    mb = lax.dot_general(mean, m2, dnt, preferred_element_type=jnp.float32)
    ib = lax.dot_general(inv, m2, dnt, preferred_element_type=jnp.float32)
    y = (x - mb) * ib * g_ref[...] + b_ref[...]
    hi = lax.broadcasted_iota(jnp.int32, (HIDDEN, 128), 0)
    ji = lax.broadcasted_iota(jnp.int32, (HIDDEN, 128), 1)
    se = (ji == hi).astype(jnp.float32)
    so = (ji == hi + HIDDEN).astype(jnp.float32)
    oute = lax.dot_general(se, y, dnt, preferred_element_type=jnp.float32)
    outo = lax.dot_general(so, y, dnt, preferred_element_type=jnp.float32)
    o_ref[...] = jnp.concatenate([oute, outo], axis=1)[None]


def _make_ln_t(chunk, aliased):
    in_specs = [
        pl.BlockSpec((_BB, 128), lambda s: (s, 0)),
        pl.BlockSpec((128,), lambda s: (0,)),
        pl.BlockSpec((128,), lambda s: (0,)),
    ]
    kwargs = {}
    if aliased:
        in_specs.append(pl.BlockSpec((1, 8, 128), lambda s: (0, 0, 0)))
        kwargs["input_output_aliases"] = {3: 0}

    def body(x_ref, g_ref, b_ref, *rest):
        _ln_t_body(x_ref, g_ref, b_ref, rest[-1])

    return pl.pallas_call(
        body,
        grid=(S_CHUNK,),
        in_specs=in_specs,
        out_specs=pl.BlockSpec(
            (1, HIDDEN, 4096), lambda s, c=chunk: (s + c * S_CHUNK, 0, 0)
        ),
        out_shape=jax.ShapeDtypeStruct((200, HIDDEN, 4096), jnp.float32),
        **kwargs,
    )


_ln_chunks = [_make_ln_t(c, aliased=(c > 0)) for c in range(CHUNKS)]


def kernel(input_ids, table, gamma, beta):
    B, S = input_ids.shape
    idx_t = (
        input_ids.astype(jnp.int32)
        .reshape(2, B // 2, S)
        .transpose(2, 1, 0)
        .reshape(-1)
    )
    g2 = jnp.concatenate([gamma, gamma])
    b2 = jnp.concatenate([beta, beta])
    tp = _pack(table.T)
    rows = [
        _sc_gather(
            lax.slice(idx_t, (c * B_CHUNK,), ((c + 1) * B_CHUNK,)).reshape(
                B_CHUNK // CH, CH
            ),
            tp,
        )
        for c in range(CHUNKS)
    ]
    rows2 = [r.reshape(B_CHUNK // 2, 128) for r in rows]
    out3 = _ln_chunks[0](rows2[0], g2, b2)
    for c in range(1, CHUNKS):
        out3 = _ln_chunks[c](rows2[c], g2, b2, out3)
    return out3.transpose(2, 0, 1)


# speedup vs baseline: 2.9462x; 1.0971x over previous
import functools

import jax
import jax.numpy as jnp
from jax import lax
from jax.experimental import pallas as pl
from jax.experimental.pallas import tpu as pltpu
from jax.experimental.pallas import tpu_sc as plsc

VOCAB = 1000000
HIDDEN = 64
EPS = 1e-12

NC = 2
NS = 16
NW = NC * NS

CH = 128
K = 2
ROWS_STEP = CH * K
B_TOTAL = 4096 * 200
CHUNKS = 2
B_CHUNK = B_TOTAL // CHUNKS
S_CHUNK = 200 // CHUNKS
B_PER_W = B_CHUNK // NW
N_STEPS = B_PER_W // ROWS_STEP

_mesh = plsc.VectorSubcoreMesh(core_axis_name="c", subcore_axis_name="s")


@functools.partial(
    pl.kernel,
    mesh=_mesh,
    compiler_params=pltpu.CompilerParams(use_tc_tiling_on_sc=False),
    out_type=jax.ShapeDtypeStruct((B_CHUNK, HIDDEN), jnp.float32),
    scratch_types=[
        pltpu.VMEM((K, CH), jnp.int32),
        pltpu.VMEM((K, CH), jnp.int32),
        pltpu.VMEM((K, CH), jnp.int32),
        pltpu.VMEM((K, CH), jnp.int32),
        pltpu.VMEM((ROWS_STEP, 2 * HIDDEN), jnp.float32),
        pltpu.VMEM((ROWS_STEP, 2 * HIDDEN), jnp.float32),
        pltpu.VMEM((ROWS_STEP, HIDDEN), jnp.float32),
        pltpu.VMEM((ROWS_STEP, HIDDEN), jnp.float32),
        pltpu.SemaphoreType.DMA,
        pltpu.SemaphoreType.DMA,
        pltpu.SemaphoreType.DMA,
        pltpu.SemaphoreType.DMA,
    ],
)
def _sc_gather(idxa_hbm, idxb_hbm, tp_hbm, out_hbm,
               idx0, idx1, pidx0, pidx1, rows2_0, rows2_1, rv0, rv1,
               semg0, semg1, semo0, semo1):
    wid = lax.axis_index("s") * NC + lax.axis_index("c")
    row0 = wid * N_STEPS
    bufs = ((idx0, pidx0, rows2_0, rv0, semg0, semo0),
            (idx1, pidx1, rows2_1, rv1, semg1, semo1))

    def issue(j, p):
        idx_v, pidx_v, rows2_v, _, semg, _ = bufs[p]
        r = row0 + j
        pltpu.sync_copy(idxa_hbm.at[pl.ds(r, 1)], idx_v.at[pl.ds(0, 1)])
        pltpu.sync_copy(idxb_hbm.at[pl.ds(r, 1)], idx_v.at[pl.ds(1, 1)])
        for k in range(2):
            for i in range(CH // 16):
                v = idx_v[k, pl.ds(16 * i, 16)]
                pidx_v[k, pl.ds(16 * i, 16)] = v >> 1
        for k in range(2):
            pltpu.async_copy(
                tp_hbm.at[pidx_v.at[k]],
                rows2_v.at[pl.ds(k * CH, CH)],
                semg,
            )

    def wait_gathers(p):
        _, pidx_v, rows2_v, _, semg, _ = bufs[p]
        for k in range(2):
            pltpu.make_async_copy(
                tp_hbm.at[pidx_v.at[k]],
                rows2_v.at[pl.ds(k * CH, CH)],
                semg,
            ).wait()

    def wait_out(j, p):
        _, _, _, rows_v, _, semo = bufs[p]
        r = row0 + j
        pltpu.make_async_copy(
            rows_v, out_hbm.at[pl.ds(r * ROWS_STEP, ROWS_STEP)], semo
        ).wait()

    def select_and_out(j, p):
        idx_v, _, rows2_v, rows_v, _, semo = bufs[p]
        r = row0 + j
        def group(i, _):
            parfa = (idx_v[0, pl.ds(16 * i, 16)] & 1).astype(jnp.float32)
            parfb = (idx_v[1, pl.ds(16 * i, 16)] & 1).astype(jnp.float32)
            for l in range(16):
                src = i * 16 + l
                for half, parf in ((0, parfa), (1, parfb)):
                    rr = 32 * i + 2 * l + half
                    hf = jnp.take(parf, jnp.full((16,), l, jnp.int32))
                    for k2 in range(4):
                        lo = rows2_v[src + 128 * half, pl.ds(16 * k2, 16)]
                        hi = rows2_v[
                            src + 128 * half, pl.ds(HIDDEN + 16 * k2, 16)
                        ]
                        rows_v[rr, pl.ds(16 * k2, 16)] = lo + (hi - lo) * hf
            return 0

        lax.fori_loop(0, CH // 16, group, 0)
        pltpu.async_copy(
            rows_v, out_hbm.at[pl.ds(r * ROWS_STEP, ROWS_STEP)], semo
        )

    issue(0, 0)

    def body(t, _):
        j0 = 2 * t
        issue(j0 + 1, 1)
        wait_gathers(0)

        @pl.when(t >= 1)
        def _():
            wait_out(j0, 0)

        select_and_out(j0, 0)

        @pl.when(t + 1 < N_STEPS // 2)
        def _():
            issue(j0 + 2, 0)

        wait_gathers(1)

        @pl.when(t >= 1)
        def _():
            wait_out(j0 + 1, 1)

        select_and_out(j0 + 1, 1)
        return 0

    lax.fori_loop(0, N_STEPS // 2, body, 0)
    wait_out(0, 0)
    wait_out(0, 1)


PACK_W = 4096
PACK_BLOCKS = -(-VOCAB // PACK_W)
TP_ROWS = PACK_BLOCKS * (PACK_W // 2)


def _pack_body(x_ref, o_ref):
    x = x_ref[...]
    r2 = lax.broadcasted_iota(jnp.int32, (128, 256), 0)
    b2 = lax.broadcasted_iota(jnp.int32, (128, 256), 1)
    se = (b2 == 2 * r2).astype(jnp.float32)
    so = (b2 == 2 * r2 + 1).astype(jnp.float32)
    dn = (((1,), (1,)), ((), ()))
    for k in range(PACK_W // 256):
        xk = x[:, 256 * k:256 * (k + 1)]
        et = lax.dot_general(se, xk, dn, preferred_element_type=jnp.float32)
        ot = lax.dot_general(so, xk, dn, preferred_element_type=jnp.float32)
        o_ref[pl.ds(128 * k, 128), :] = jnp.concatenate([et, ot], axis=1)


_pack = pl.pallas_call(
    _pack_body,
    grid=(PACK_BLOCKS,),
    in_specs=[pl.BlockSpec((HIDDEN, PACK_W), lambda b: (0, b))],
    out_specs=pl.BlockSpec((PACK_W // 2, 128), lambda b: (b, 0)),
    out_shape=jax.ShapeDtypeStruct((TP_ROWS, 128), jnp.float32),
)


_BB = 2048


def _ln_t_body(x_ref, g_ref, b_ref, o_ref):
    x = x_ref[...]
    dn = (((1,), (0,)), ((), ()))
    dnt = (((1,), (1,)), ((), ()))
    jj = lax.broadcasted_iota(jnp.int32, (128, 2), 0)
    hh = lax.broadcasted_iota(jnp.int32, (128, 2), 1)
    m2 = ((jj // HIDDEN) == hh).astype(jnp.float32)
    s = lax.dot_general(x, m2, dn, preferred_element_type=jnp.float32)
    sq = lax.dot_general(x * x, m2, dn, preferred_element_type=jnp.float32)
    mean = s * (1.0 / HIDDEN)
    var = sq * (1.0 / HIDDEN) - mean * mean
    inv = lax.rsqrt(var + EPS)
    mb = lax.dot_general(mean, m2, dnt, preferred_element_type=jnp.float32)
    ib = lax.dot_general(inv, m2, dnt, preferred_element_type=jnp.float32)
    y = (x - mb) * ib * g_ref[...] + b_ref[...]
    hi = lax.broadcasted_iota(jnp.int32, (HIDDEN, 128), 0)
    ji = lax.broadcasted_iota(jnp.int32, (HIDDEN, 128), 1)
    se = (ji == hi).astype(jnp.float32)
    so = (ji == hi + HIDDEN).astype(jnp.float32)
    oute = lax.dot_general(se, y, dnt, preferred_element_type=jnp.float32)
    outo = lax.dot_general(so, y, dnt, preferred_element_type=jnp.float32)
    o_ref[...] = jnp.concatenate([oute, outo], axis=1)[None]


def _make_ln_t(chunk, aliased):
    in_specs = [
        pl.BlockSpec((_BB, 128), lambda s: (s, 0)),
        pl.BlockSpec((128,), lambda s: (0,)),
        pl.BlockSpec((128,), lambda s: (0,)),
    ]
    kwargs = {}
    if aliased:
        in_specs.append(pl.BlockSpec((1, 8, 128), lambda s: (0, 0, 0)))
        kwargs["input_output_aliases"] = {3: 0}

    def body(x_ref, g_ref, b_ref, *rest):
        _ln_t_body(x_ref, g_ref, b_ref, rest[-1])

    return pl.pallas_call(
        body,
        grid=(S_CHUNK,),
        in_specs=in_specs,
        out_specs=pl.BlockSpec(
            (1, HIDDEN, 4096), lambda s, c=chunk: (s + c * S_CHUNK, 0, 0)
        ),
        out_shape=jax.ShapeDtypeStruct((200, HIDDEN, 4096), jnp.float32),
        **kwargs,
    )


_ln_chunks = [_make_ln_t(c, aliased=(c > 0)) for c in range(CHUNKS)]


def kernel(input_ids, table, gamma, beta):
    B, S = input_ids.shape
    ids_t = input_ids.T.astype(jnp.int32)
    idxa = ids_t[:, : B // 2].reshape(-1)
    idxb = ids_t[:, B // 2:].reshape(-1)
    g2 = jnp.concatenate([gamma, gamma])
    b2 = jnp.concatenate([beta, beta])
    tp = _pack(table.T)
    hc = B_CHUNK // 2
    rows = [
        _sc_gather(
            lax.slice(idxa, (c * hc,), ((c + 1) * hc,)).reshape(hc // CH, CH),
            lax.slice(idxb, (c * hc,), ((c + 1) * hc,)).reshape(hc // CH, CH),
            tp,
        )
        for c in range(CHUNKS)
    ]
    rows2 = [r.reshape(B_CHUNK // 2, 128) for r in rows]
    out3 = _ln_chunks[0](rows2[0], g2, b2)
    for c in range(1, CHUNKS):
        out3 = _ln_chunks[c](rows2[c], g2, b2, out3)
    return out3.transpose(2, 0, 1)


# speedup vs baseline: 3.2694x; 1.1097x over previous
import functools

import jax
import jax.numpy as jnp
from jax import lax
from jax.experimental import pallas as pl
from jax.experimental.pallas import tpu as pltpu
from jax.experimental.pallas import tpu_sc as plsc

VOCAB = 1000000
HIDDEN = 64
EPS = 1e-12

NC = 2
NS = 16
NW = NC * NS

CH = 128
K = 2
ROWS_STEP = CH * K
B_TOTAL = 4096 * 200
CHUNKS = 2
B_CHUNK = B_TOTAL // CHUNKS
S_CHUNK = 200 // CHUNKS
B_PER_W = B_CHUNK // NW
N_STEPS = B_PER_W // ROWS_STEP

_mesh = plsc.VectorSubcoreMesh(core_axis_name="c", subcore_axis_name="s")


@functools.partial(
    pl.kernel,
    mesh=_mesh,
    compiler_params=pltpu.CompilerParams(use_tc_tiling_on_sc=False),
    out_type=jax.ShapeDtypeStruct((B_CHUNK, HIDDEN), jnp.float32),
    scratch_types=[
        pltpu.VMEM((K, CH), jnp.int32),
        pltpu.VMEM((K, CH), jnp.int32),
        pltpu.VMEM((K, CH), jnp.int32),
        pltpu.VMEM((K, CH), jnp.int32),
        pltpu.VMEM((ROWS_STEP, 2 * HIDDEN), jnp.float32),
        pltpu.VMEM((ROWS_STEP, 2 * HIDDEN), jnp.float32),
        pltpu.VMEM((ROWS_STEP, HIDDEN), jnp.float32),
        pltpu.VMEM((ROWS_STEP, HIDDEN), jnp.float32),
        pltpu.VMEM((2, CH), jnp.float32),
        pltpu.VMEM((2, CH), jnp.float32),
        pltpu.SemaphoreType.DMA,
        pltpu.SemaphoreType.DMA,
        pltpu.SemaphoreType.DMA,
        pltpu.SemaphoreType.DMA,
        pltpu.SemaphoreType.DMA,
        pltpu.SemaphoreType.DMA,
    ],
)
def _sc_gather(idxa_hbm, idxb_hbm, tp_hbm, out_hbm,
               idx0, idx1, pidx0, pidx1, rows2_0, rows2_1, rv0, rv1,
               parf0, parf1, semg0, semg1, semo0, semo1, semi0, semi1):
    wid = lax.axis_index("s") * NC + lax.axis_index("c")
    row0 = wid * N_STEPS
    bufs = ((idx0, pidx0, rows2_0, rv0, parf0, semg0, semo0, semi0),
            (idx1, pidx1, rows2_1, rv1, parf1, semg1, semo1, semi1))

    def prefetch_idx(j, p):
        idx_v, _, _, _, _, _, _, semi = bufs[p]
        r = row0 + j
        pltpu.async_copy(idxa_hbm.at[pl.ds(r, 1)], idx_v.at[pl.ds(0, 1)], semi)
        pltpu.async_copy(idxb_hbm.at[pl.ds(r, 1)], idx_v.at[pl.ds(1, 1)], semi)

    def issue(j, p):
        idx_v, pidx_v, rows2_v, _, parf_v, semg, _, semi = bufs[p]
        r = row0 + j
        pltpu.make_async_copy(
            idxa_hbm.at[pl.ds(r, 1)], idx_v.at[pl.ds(0, 1)], semi
        ).wait()
        pltpu.make_async_copy(
            idxb_hbm.at[pl.ds(r, 1)], idx_v.at[pl.ds(1, 1)], semi
        ).wait()
        for k in range(2):
            for i in range(CH // 16):
                v = idx_v[k, pl.ds(16 * i, 16)]
                pidx_v[k, pl.ds(16 * i, 16)] = v >> 1
                parf_v[k, pl.ds(16 * i, 16)] = (v & 1).astype(jnp.float32)
        for k in range(2):
            pltpu.async_copy(
                tp_hbm.at[pidx_v.at[k]],
                rows2_v.at[pl.ds(k * CH, CH)],
                semg,
            )

    def wait_gathers(p):
        _, pidx_v, rows2_v, _, _, semg, _, _ = bufs[p]
        for k in range(2):
            pltpu.make_async_copy(
                tp_hbm.at[pidx_v.at[k]],
                rows2_v.at[pl.ds(k * CH, CH)],
                semg,
            ).wait()

    def wait_out(j, p):
        _, _, _, rows_v, _, _, semo, _ = bufs[p]
        r = row0 + j
        pltpu.make_async_copy(
            rows_v, out_hbm.at[pl.ds(r * ROWS_STEP, ROWS_STEP)], semo
        ).wait()

    def select_and_out(j, p):
        _, _, rows2_v, rows_v, parf_v, _, semo, _ = bufs[p]
        r = row0 + j
        def group(i, _):
            parfa = parf_v[0, pl.ds(16 * i, 16)]
            parfb = parf_v[1, pl.ds(16 * i, 16)]
            for l in range(16):
                src = i * 16 + l
                for half, parf in ((0, parfa), (1, parfb)):
                    rr = 32 * i + 2 * l + half
                    hf = jnp.take(parf, jnp.full((16,), l, jnp.int32))
                    for k2 in range(4):
                        lo = rows2_v[src + 128 * half, pl.ds(16 * k2, 16)]
                        hi = rows2_v[
                            src + 128 * half, pl.ds(HIDDEN + 16 * k2, 16)
                        ]
                        rows_v[rr, pl.ds(16 * k2, 16)] = lo + (hi - lo) * hf
            return 0

        lax.fori_loop(0, CH // 16, group, 0)
        pltpu.async_copy(
            rows_v, out_hbm.at[pl.ds(r * ROWS_STEP, ROWS_STEP)], semo
        )

    prefetch_idx(0, 0)
    issue(0, 0)
    prefetch_idx(1, 1)

    def body(t, _):
        j0 = 2 * t
        issue(j0 + 1, 1)

        @pl.when(t + 1 < N_STEPS // 2)
        def _():
            prefetch_idx(j0 + 2, 0)

        wait_gathers(0)

        @pl.when(t >= 1)
        def _():
            wait_out(j0, 0)

        select_and_out(j0, 0)

        @pl.when(t + 1 < N_STEPS // 2)
        def _():
            issue(j0 + 2, 0)
            prefetch_idx(j0 + 3, 1)

        wait_gathers(1)

        @pl.when(t >= 1)
        def _():
            wait_out(j0 + 1, 1)

        select_and_out(j0 + 1, 1)
        return 0

    lax.fori_loop(0, N_STEPS // 2, body, 0)
    wait_out(0, 0)
    wait_out(0, 1)


PACK_W = 4096
PACK_BLOCKS = -(-VOCAB // PACK_W)
TP_ROWS = PACK_BLOCKS * (PACK_W // 2)


def _pack_body(x_ref, o_ref):
    x = x_ref[...]
    r2 = lax.broadcasted_iota(jnp.int32, (128, 256), 0)
    b2 = lax.broadcasted_iota(jnp.int32, (128, 256), 1)
    se = (b2 == 2 * r2).astype(jnp.float32)
    so = (b2 == 2 * r2 + 1).astype(jnp.float32)
    dn = (((1,), (1,)), ((), ()))
    for k in range(PACK_W // 256):
        xk = x[:, 256 * k:256 * (k + 1)]
        et = lax.dot_general(se, xk, dn, preferred_element_type=jnp.float32)
        ot = lax.dot_general(so, xk, dn, preferred_element_type=jnp.float32)
        o_ref[pl.ds(128 * k, 128), :] = jnp.concatenate([et, ot], axis=1)


_pack = pl.pallas_call(
    _pack_body,
    grid=(PACK_BLOCKS,),
    in_specs=[pl.BlockSpec((HIDDEN, PACK_W), lambda b: (0, b))],
    out_specs=pl.BlockSpec((PACK_W // 2, 128), lambda b: (b, 0)),
    out_shape=jax.ShapeDtypeStruct((TP_ROWS, 128), jnp.float32),
)


_BB = 2048


def _ln_t_body(x_ref, g_ref, b_ref, o_ref):
    x = x_ref[...]
    dn = (((1,), (0,)), ((), ()))
    dnt = (((1,), (1,)), ((), ()))
    jj = lax.broadcasted_iota(jnp.int32, (128, 2), 0)
    hh = lax.broadcasted_iota(jnp.int32, (128, 2), 1)
    m2 = ((jj // HIDDEN) == hh).astype(jnp.float32)
    s = lax.dot_general(x, m2, dn, preferred_element_type=jnp.float32)
    sq = lax.dot_general(x * x, m2, dn, preferred_element_type=jnp.float32)
    mean = s * (1.0 / HIDDEN)
    var = sq * (1.0 / HIDDEN) - mean * mean
    inv = lax.rsqrt(var + EPS)
    mb = lax.dot_general(mean, m2, dnt, preferred_element_type=jnp.float32)
    ib = lax.dot_general(inv, m2, dnt, preferred_element_type=jnp.float32)
    y = (x - mb) * ib * g_ref[...] + b_ref[...]
    hi = lax.broadcasted_iota(jnp.int32, (HIDDEN, 128), 0)
    ji = lax.broadcasted_iota(jnp.int32, (HIDDEN, 128), 1)
    se = (ji == hi).astype(jnp.float32)
    so = (ji == hi + HIDDEN).astype(jnp.float32)
    oute = lax.dot_general(se, y, dnt, preferred_element_type=jnp.float32)
    outo = lax.dot_general(so, y, dnt, preferred_element_type=jnp.float32)
    o_ref[...] = jnp.concatenate([oute, outo], axis=1)[None]


def _make_ln_t(chunk, aliased):
    in_specs = [
        pl.BlockSpec((_BB, 128), lambda s: (s, 0)),
        pl.BlockSpec((128,), lambda s: (0,)),
        pl.BlockSpec((128,), lambda s: (0,)),
    ]
    kwargs = {}
    if aliased:
        in_specs.append(pl.BlockSpec((1, 8, 128), lambda s: (0, 0, 0)))
        kwargs["input_output_aliases"] = {3: 0}

    def body(x_ref, g_ref, b_ref, *rest):
        _ln_t_body(x_ref, g_ref, b_ref, rest[-1])

    return pl.pallas_call(
        body,
        grid=(S_CHUNK,),
        in_specs=in_specs,
        out_specs=pl.BlockSpec(
            (1, HIDDEN, 4096), lambda s, c=chunk: (s + c * S_CHUNK, 0, 0)
        ),
        out_shape=jax.ShapeDtypeStruct((200, HIDDEN, 4096), jnp.float32),
        **kwargs,
    )


_ln_chunks = [_make_ln_t(c, aliased=(c > 0)) for c in range(CHUNKS)]


def kernel(input_ids, table, gamma, beta):
    B, S = input_ids.shape
    ids_t = input_ids.T.astype(jnp.int32)
    idxa = ids_t[:, : B // 2].reshape(-1)
    idxb = ids_t[:, B // 2:].reshape(-1)
    g2 = jnp.concatenate([gamma, gamma])
    b2 = jnp.concatenate([beta, beta])
    tp = _pack(table.T)
    hc = B_CHUNK // 2
    rows = [
        _sc_gather(
            lax.slice(idxa, (c * hc,), ((c + 1) * hc,)).reshape(hc // CH, CH),
            lax.slice(idxb, (c * hc,), ((c + 1) * hc,)).reshape(hc // CH, CH),
            tp,
        )
        for c in range(CHUNKS)
    ]
    rows2 = [r.reshape(B_CHUNK // 2, 128) for r in rows]
    out3 = _ln_chunks[0](rows2[0], g2, b2)
    for c in range(1, CHUNKS):
        out3 = _ln_chunks[c](rows2[c], g2, b2, out3)
    return out3.transpose(2, 0, 1)


# speedup vs baseline: 3.5368x; 1.0818x over previous
import functools

import jax
import jax.numpy as jnp
from jax import lax
from jax.experimental import pallas as pl
from jax.experimental.pallas import tpu as pltpu
from jax.experimental.pallas import tpu_sc as plsc

VOCAB = 1000000
HIDDEN = 64
EPS = 1e-12

NC = 2
NS = 16
NW = NC * NS

CH = 128
K = 2
ROWS_STEP = CH * K
B_TOTAL = 4096 * 200
CHUNKS = 5
B_CHUNK = B_TOTAL // CHUNKS
S_CHUNK = 200 // CHUNKS
B_PER_W = B_CHUNK // NW
N_STEPS = B_PER_W // ROWS_STEP

_mesh = plsc.VectorSubcoreMesh(core_axis_name="c", subcore_axis_name="s")


@functools.partial(
    pl.kernel,
    mesh=_mesh,
    compiler_params=pltpu.CompilerParams(use_tc_tiling_on_sc=False),
    out_type=jax.ShapeDtypeStruct((B_CHUNK, HIDDEN), jnp.float32),
    scratch_types=[
        pltpu.VMEM((K, CH), jnp.int32),
        pltpu.VMEM((K, CH), jnp.int32),
        pltpu.VMEM((K, CH), jnp.int32),
        pltpu.VMEM((K, CH), jnp.int32),
        pltpu.VMEM((ROWS_STEP, 2 * HIDDEN), jnp.float32),
        pltpu.VMEM((ROWS_STEP, 2 * HIDDEN), jnp.float32),
        pltpu.VMEM((ROWS_STEP, HIDDEN), jnp.float32),
        pltpu.VMEM((ROWS_STEP, HIDDEN), jnp.float32),
        pltpu.VMEM((2, CH), jnp.float32),
        pltpu.VMEM((2, CH), jnp.float32),
        pltpu.SemaphoreType.DMA,
        pltpu.SemaphoreType.DMA,
        pltpu.SemaphoreType.DMA,
        pltpu.SemaphoreType.DMA,
        pltpu.SemaphoreType.DMA,
        pltpu.SemaphoreType.DMA,
    ],
)
def _sc_gather(idxa_hbm, idxb_hbm, tp_hbm, out_hbm,
               idx0, idx1, pidx0, pidx1, rows2_0, rows2_1, rv0, rv1,
               parf0, parf1, semg0, semg1, semo0, semo1, semi0, semi1):
    wid = lax.axis_index("s") * NC + lax.axis_index("c")
    row0 = wid * N_STEPS
    bufs = ((idx0, pidx0, rows2_0, rv0, parf0, semg0, semo0, semi0),
            (idx1, pidx1, rows2_1, rv1, parf1, semg1, semo1, semi1))

    def prefetch_idx(j, p):
        idx_v, _, _, _, _, _, _, semi = bufs[p]
        r = row0 + j
        pltpu.async_copy(idxa_hbm.at[pl.ds(r, 1)], idx_v.at[pl.ds(0, 1)], semi)
        pltpu.async_copy(idxb_hbm.at[pl.ds(r, 1)], idx_v.at[pl.ds(1, 1)], semi)

    def issue(j, p):
        idx_v, pidx_v, rows2_v, _, parf_v, semg, _, semi = bufs[p]
        r = row0 + j
        pltpu.make_async_copy(
            idxa_hbm.at[pl.ds(r, 1)], idx_v.at[pl.ds(0, 1)], semi
        ).wait()
        pltpu.make_async_copy(
            idxb_hbm.at[pl.ds(r, 1)], idx_v.at[pl.ds(1, 1)], semi
        ).wait()
        for k in range(2):
            for i in range(CH // 16):
                v = idx_v[k, pl.ds(16 * i, 16)]
                pidx_v[k, pl.ds(16 * i, 16)] = v >> 1
                parf_v[k, pl.ds(16 * i, 16)] = (v & 1).astype(jnp.float32)
        for k in range(2):
            pltpu.async_copy(
                tp_hbm.at[pidx_v.at[k]],
                rows2_v.at[pl.ds(k * CH, CH)],
                semg,
            )

    def wait_gathers(p):
        _, pidx_v, rows2_v, _, _, semg, _, _ = bufs[p]
        for k in range(2):
            pltpu.make_async_copy(
                tp_hbm.at[pidx_v.at[k]],
                rows2_v.at[pl.ds(k * CH, CH)],
                semg,
            ).wait()

    def wait_out(j, p):
        _, _, _, rows_v, _, _, semo, _ = bufs[p]
        r = row0 + j
        pltpu.make_async_copy(
            rows_v, out_hbm.at[pl.ds(r * ROWS_STEP, ROWS_STEP)], semo
        ).wait()

    def select_and_out(j, p):
        _, _, rows2_v, rows_v, parf_v, _, semo, _ = bufs[p]
        r = row0 + j
        def group(i, _):
            parfa = parf_v[0, pl.ds(16 * i, 16)]
            parfb = parf_v[1, pl.ds(16 * i, 16)]
            for l in range(16):
                src = i * 16 + l
                for half, parf in ((0, parfa), (1, parfb)):
                    rr = 32 * i + 2 * l + half
                    hf = jnp.take(parf, jnp.full((16,), l, jnp.int32))
                    for k2 in range(4):
                        lo = rows2_v[src + 128 * half, pl.ds(16 * k2, 16)]
                        hi = rows2_v[
                            src + 128 * half, pl.ds(HIDDEN + 16 * k2, 16)
                        ]
                        rows_v[rr, pl.ds(16 * k2, 16)] = lo + (hi - lo) * hf
            return 0

        lax.fori_loop(0, CH // 16, group, 0)
        pltpu.async_copy(
            rows_v, out_hbm.at[pl.ds(r * ROWS_STEP, ROWS_STEP)], semo
        )

    prefetch_idx(0, 0)
    issue(0, 0)
    prefetch_idx(1, 1)

    def body(t, _):
        j0 = 2 * t
        issue(j0 + 1, 1)

        @pl.when(t + 1 < N_STEPS // 2)
        def _():
            prefetch_idx(j0 + 2, 0)

        wait_gathers(0)

        @pl.when(t >= 1)
        def _():
            wait_out(j0, 0)

        select_and_out(j0, 0)

        @pl.when(t + 1 < N_STEPS // 2)
        def _():
            issue(j0 + 2, 0)
            prefetch_idx(j0 + 3, 1)

        wait_gathers(1)

        @pl.when(t >= 1)
        def _():
            wait_out(j0 + 1, 1)

        select_and_out(j0 + 1, 1)
        return 0

    lax.fori_loop(0, N_STEPS // 2, body, 0)
    wait_out(0, 0)
    wait_out(0, 1)


PACK_W = 8192
PACK_BLOCKS = -(-VOCAB // PACK_W)
TP_ROWS = PACK_BLOCKS * (PACK_W // 2)


def _pack_body(x_ref, o_ref):
    x = x_ref[...]
    r2 = lax.broadcasted_iota(jnp.int32, (128, 256), 0)
    b2 = lax.broadcasted_iota(jnp.int32, (128, 256), 1)
    se = (b2 == 2 * r2).astype(jnp.float32)
    so = (b2 == 2 * r2 + 1).astype(jnp.float32)
    dn = (((1,), (1,)), ((), ()))
    for k in range(PACK_W // 256):
        xk = x[:, 256 * k:256 * (k + 1)]
        et = lax.dot_general(se, xk, dn, preferred_element_type=jnp.float32)
        ot = lax.dot_general(so, xk, dn, preferred_element_type=jnp.float32)
        o_ref[pl.ds(128 * k, 128), :] = jnp.concatenate([et, ot], axis=1)


_pack = pl.pallas_call(
    _pack_body,
    grid=(PACK_BLOCKS,),
    in_specs=[pl.BlockSpec((HIDDEN, PACK_W), lambda b: (0, b))],
    out_specs=pl.BlockSpec((PACK_W // 2, 128), lambda b: (b, 0)),
    out_shape=jax.ShapeDtypeStruct((TP_ROWS, 128), jnp.float32),
)


_BB = 2048


def _ln_t_body(x_ref, g_ref, b_ref, o_ref):
    x = x_ref[...]
    dn = (((1,), (0,)), ((), ()))
    dnt = (((1,), (1,)), ((), ()))
    jj = lax.broadcasted_iota(jnp.int32, (128, 2), 0)
    hh = lax.broadcasted_iota(jnp.int32, (128, 2), 1)
    m2 = ((jj // HIDDEN) == hh).astype(jnp.float32)
    s = lax.dot_general(x, m2, dn, preferred_element_type=jnp.float32)
    sq = lax.dot_general(x * x, m2, dn, preferred_element_type=jnp.float32)
    mean = s * (1.0 / HIDDEN)
    var = sq * (1.0 / HIDDEN) - mean * mean
    inv = lax.rsqrt(var + EPS)
    mb = lax.dot_general(mean, m2, dnt, preferred_element_type=jnp.float32)
    ib = lax.dot_general(inv, m2, dnt, preferred_element_type=jnp.float32)
    y = (x - mb) * ib * g_ref[...] + b_ref[...]
    hi = lax.broadcasted_iota(jnp.int32, (HIDDEN, 128), 0)
    ji = lax.broadcasted_iota(jnp.int32, (HIDDEN, 128), 1)
    se = (ji == hi).astype(jnp.float32)
    so = (ji == hi + HIDDEN).astype(jnp.float32)
    oute = lax.dot_general(se, y, dnt, preferred_element_type=jnp.float32)
    outo = lax.dot_general(so, y, dnt, preferred_element_type=jnp.float32)
    o_ref[...] = jnp.concatenate([oute, outo], axis=1)[None]


def _make_ln_t(chunk, aliased):
    in_specs = [
        pl.BlockSpec((_BB, 128), lambda s: (s, 0)),
        pl.BlockSpec((128,), lambda s: (0,)),
        pl.BlockSpec((128,), lambda s: (0,)),
    ]
    kwargs = {}
    if aliased:
        in_specs.append(pl.BlockSpec((1, 8, 128), lambda s: (0, 0, 0)))
        kwargs["input_output_aliases"] = {3: 0}

    def body(x_ref, g_ref, b_ref, *rest):
        _ln_t_body(x_ref, g_ref, b_ref, rest[-1])

    return pl.pallas_call(
        body,
        grid=(S_CHUNK,),
        in_specs=in_specs,
        out_specs=pl.BlockSpec(
            (1, HIDDEN, 4096), lambda s, c=chunk: (s + c * S_CHUNK, 0, 0)
        ),
        out_shape=jax.ShapeDtypeStruct((200, HIDDEN, 4096), jnp.float32),
        **kwargs,
    )


_ln_chunks = [_make_ln_t(c, aliased=(c > 0)) for c in range(CHUNKS)]


def kernel(input_ids, table, gamma, beta):
    B, S = input_ids.shape
    ids_t = input_ids.T.astype(jnp.int32)
    idxa = ids_t[:, : B // 2].reshape(-1)
    idxb = ids_t[:, B // 2:].reshape(-1)
    g2 = jnp.concatenate([gamma, gamma])
    b2 = jnp.concatenate([beta, beta])
    tp = _pack(table.T)
    hc = B_CHUNK // 2
    rows = [
        _sc_gather(
            lax.slice(idxa, (c * hc,), ((c + 1) * hc,)).reshape(hc // CH, CH),
            lax.slice(idxb, (c * hc,), ((c + 1) * hc,)).reshape(hc // CH, CH),
            tp,
        )
        for c in range(CHUNKS)
    ]
    rows2 = [r.reshape(B_CHUNK // 2, 128) for r in rows]
    out3 = _ln_chunks[0](rows2[0], g2, b2)
    for c in range(1, CHUNKS):
        out3 = _ln_chunks[c](rows2[c], g2, b2, out3)
    return out3.transpose(2, 0, 1)


# speedup vs baseline: 3.5374x; 1.0002x over previous
import functools

import jax
import jax.numpy as jnp
from jax import lax
from jax.experimental import pallas as pl
from jax.experimental.pallas import tpu as pltpu
from jax.experimental.pallas import tpu_sc as plsc

VOCAB = 1000000
HIDDEN = 64
EPS = 1e-12

NC = 2
NS = 16
NW = NC * NS

CH = 128
K = 2
ROWS_STEP = CH * K
B_TOTAL = 4096 * 200
CHUNKS = 5
B_CHUNK = B_TOTAL // CHUNKS
S_CHUNK = 200 // CHUNKS
B_PER_W = B_CHUNK // NW
N_STEPS = B_PER_W // ROWS_STEP

_mesh = plsc.VectorSubcoreMesh(core_axis_name="c", subcore_axis_name="s")


@functools.partial(
    pl.kernel,
    mesh=_mesh,
    compiler_params=pltpu.CompilerParams(use_tc_tiling_on_sc=False),
    out_type=jax.ShapeDtypeStruct((B_CHUNK, HIDDEN), jnp.float32),
    scratch_types=[
        pltpu.VMEM((K, CH), jnp.int32),
        pltpu.VMEM((K, CH), jnp.int32),
        pltpu.VMEM((K, CH), jnp.int32),
        pltpu.VMEM((K, CH), jnp.int32),
        pltpu.VMEM((ROWS_STEP, 2 * HIDDEN), jnp.float32),
        pltpu.VMEM((ROWS_STEP, 2 * HIDDEN), jnp.float32),
        pltpu.VMEM((ROWS_STEP, HIDDEN), jnp.float32),
        pltpu.VMEM((ROWS_STEP, HIDDEN), jnp.float32),
        pltpu.VMEM((2, CH), jnp.float32),
        pltpu.VMEM((2, CH), jnp.float32),
        pltpu.SemaphoreType.DMA,
        pltpu.SemaphoreType.DMA,
        pltpu.SemaphoreType.DMA,
        pltpu.SemaphoreType.DMA,
        pltpu.SemaphoreType.DMA,
        pltpu.SemaphoreType.DMA,
    ],
)
def _sc_gather(idxa_hbm, idxb_hbm, tp_hbm, out_hbm,
               idx0, idx1, pidx0, pidx1, rows2_0, rows2_1, rv0, rv1,
               parf0, parf1, semg0, semg1, semo0, semo1, semi0, semi1):
    wid = lax.axis_index("s") * NC + lax.axis_index("c")
    row0 = wid * N_STEPS
    bufs = ((idx0, pidx0, rows2_0, rv0, parf0, semg0, semo0, semi0),
            (idx1, pidx1, rows2_1, rv1, parf1, semg1, semo1, semi1))

    def prefetch_idx(j, p):
        idx_v, _, _, _, _, _, _, semi = bufs[p]
        r = row0 + j
        pltpu.async_copy(idxa_hbm.at[pl.ds(r, 1)], idx_v.at[pl.ds(0, 1)], semi)
        pltpu.async_copy(idxb_hbm.at[pl.ds(r, 1)], idx_v.at[pl.ds(1, 1)], semi)

    def issue(j, p):
        idx_v, pidx_v, rows2_v, _, parf_v, semg, _, semi = bufs[p]
        r = row0 + j
        pltpu.make_async_copy(
            idxa_hbm.at[pl.ds(r, 1)], idx_v.at[pl.ds(0, 1)], semi
        ).wait()
        pltpu.make_async_copy(
            idxb_hbm.at[pl.ds(r, 1)], idx_v.at[pl.ds(1, 1)], semi
        ).wait()
        for k in range(2):
            for i in range(CH // 16):
                v = idx_v[k, pl.ds(16 * i, 16)]
                pidx_v[k, pl.ds(16 * i, 16)] = v >> 1
                parf_v[k, pl.ds(16 * i, 16)] = (v & 1).astype(jnp.float32)
        for k in range(2):
            for h in range(2):
                pltpu.async_copy(
                    tp_hbm.at[pidx_v.at[k, pl.ds(64 * h, 64)]],
                    rows2_v.at[pl.ds(k * CH + 64 * h, 64)],
                    semg,
                )

    def wait_gathers(p):
        _, pidx_v, rows2_v, _, _, semg, _, _ = bufs[p]
        for k in range(2):
            for h in range(2):
                pltpu.make_async_copy(
                    tp_hbm.at[pidx_v.at[k, pl.ds(64 * h, 64)]],
                    rows2_v.at[pl.ds(k * CH + 64 * h, 64)],
                    semg,
                ).wait()

    def wait_out(j, p):
        _, _, _, rows_v, _, _, semo, _ = bufs[p]
        r = row0 + j
        pltpu.make_async_copy(
            rows_v, out_hbm.at[pl.ds(r * ROWS_STEP, ROWS_STEP)], semo
        ).wait()

    def select_and_out(j, p):
        _, _, rows2_v, rows_v, parf_v, _, semo, _ = bufs[p]
        r = row0 + j
        def group(i, _):
            parfa = parf_v[0, pl.ds(16 * i, 16)]
            parfb = parf_v[1, pl.ds(16 * i, 16)]
            for l in range(16):
                src = i * 16 + l
                for half, parf in ((0, parfa), (1, parfb)):
                    rr = 32 * i + 2 * l + half
                    hf = jnp.take(parf, jnp.full((16,), l, jnp.int32))
                    for k2 in range(4):
                        lo = rows2_v[src + 128 * half, pl.ds(16 * k2, 16)]
                        hi = rows2_v[
                            src + 128 * half, pl.ds(HIDDEN + 16 * k2, 16)
                        ]
                        rows_v[rr, pl.ds(16 * k2, 16)] = lo + (hi - lo) * hf
            return 0

        lax.fori_loop(0, CH // 16, group, 0)
        pltpu.async_copy(
            rows_v, out_hbm.at[pl.ds(r * ROWS_STEP, ROWS_STEP)], semo
        )

    prefetch_idx(0, 0)
    issue(0, 0)
    prefetch_idx(1, 1)

    def body(t, _):
        j0 = 2 * t
        issue(j0 + 1, 1)

        @pl.when(t + 1 < N_STEPS // 2)
        def _():
            prefetch_idx(j0 + 2, 0)

        wait_gathers(0)

        @pl.when(t >= 1)
        def _():
            wait_out(j0, 0)

        select_and_out(j0, 0)

        @pl.when(t + 1 < N_STEPS // 2)
        def _():
            issue(j0 + 2, 0)
            prefetch_idx(j0 + 3, 1)

        wait_gathers(1)

        @pl.when(t >= 1)
        def _():
            wait_out(j0 + 1, 1)

        select_and_out(j0 + 1, 1)
        return 0

    lax.fori_loop(0, N_STEPS // 2, body, 0)
    wait_out(0, 0)
    wait_out(0, 1)


PACK_W = 8192
PACK_BLOCKS = -(-VOCAB // PACK_W)
TP_ROWS = PACK_BLOCKS * (PACK_W // 2)


def _pack_body(x_ref, o_ref):
    x = x_ref[...]
    r2 = lax.broadcasted_iota(jnp.int32, (128, 256), 0)
    b2 = lax.broadcasted_iota(jnp.int32, (128, 256), 1)
    se = (b2 == 2 * r2).astype(jnp.float32)
    so = (b2 == 2 * r2 + 1).astype(jnp.float32)
    dn = (((1,), (1,)), ((), ()))
    for k in range(PACK_W // 256):
        xk = x[:, 256 * k:256 * (k + 1)]
        et = lax.dot_general(se, xk, dn, preferred_element_type=jnp.float32)
        ot = lax.dot_general(so, xk, dn, preferred_element_type=jnp.float32)
        o_ref[pl.ds(128 * k, 128), :] = jnp.concatenate([et, ot], axis=1)


_pack = pl.pallas_call(
    _pack_body,
    grid=(PACK_BLOCKS,),
    in_specs=[pl.BlockSpec((HIDDEN, PACK_W), lambda b: (0, b))],
    out_specs=pl.BlockSpec((PACK_W // 2, 128), lambda b: (b, 0)),
    out_shape=jax.ShapeDtypeStruct((TP_ROWS, 128), jnp.float32),
)


_BB = 2048


def _ln_t_body(x_ref, g_ref, b_ref, o_ref):
    x = x_ref[...]
    dn = (((1,), (0,)), ((), ()))
    dnt = (((1,), (1,)), ((), ()))
    jj = lax.broadcasted_iota(jnp.int32, (128, 2), 0)
    hh = lax.broadcasted_iota(jnp.int32, (128, 2), 1)
    m2 = ((jj // HIDDEN) == hh).astype(jnp.float32)
    s = lax.dot_general(x, m2, dn, preferred_element_type=jnp.float32)
    sq = lax.dot_general(x * x, m2, dn, preferred_element_type=jnp.float32)
    mean = s * (1.0 / HIDDEN)
    var = sq * (1.0 / HIDDEN) - mean * mean
    inv = lax.rsqrt(var + EPS)
    mb = lax.dot_general(mean, m2, dnt, preferred_element_type=jnp.float32)
    ib = lax.dot_general(inv, m2, dnt, preferred_element_type=jnp.float32)
    y = (x - mb) * ib * g_ref[...] + b_ref[...]
    hi = lax.broadcasted_iota(jnp.int32, (HIDDEN, 128), 0)
    ji = lax.broadcasted_iota(jnp.int32, (HIDDEN, 128), 1)
    se = (ji == hi).astype(jnp.float32)
    so = (ji == hi + HIDDEN).astype(jnp.float32)
    oute = lax.dot_general(se, y, dnt, preferred_element_type=jnp.float32)
    outo = lax.dot_general(so, y, dnt, preferred_element_type=jnp.float32)
    o_ref[...] = jnp.concatenate([oute, outo], axis=1)[None]


def _make_ln_t(chunk, aliased):
    in_specs = [
        pl.BlockSpec((_BB, 128), lambda s: (s, 0)),
        pl.BlockSpec((128,), lambda s: (0,)),
        pl.BlockSpec((128,), lambda s: (0,)),
    ]
    kwargs = {}
    if aliased:
        in_specs.append(pl.BlockSpec((1, 8, 128), lambda s: (0, 0, 0)))
        kwargs["input_output_aliases"] = {3: 0}

    def body(x_ref, g_ref, b_ref, *rest):
        _ln_t_body(x_ref, g_ref, b_ref, rest[-1])

    return pl.pallas_call(
        body,
        grid=(S_CHUNK,),
        in_specs=in_specs,
        out_specs=pl.BlockSpec(
            (1, HIDDEN, 4096), lambda s, c=chunk: (s + c * S_CHUNK, 0, 0)
        ),
        out_shape=jax.ShapeDtypeStruct((200, HIDDEN, 4096), jnp.float32),
        **kwargs,
    )


_ln_chunks = [_make_ln_t(c, aliased=(c > 0)) for c in range(CHUNKS)]


def kernel(input_ids, table, gamma, beta):
    B, S = input_ids.shape
    ids_t = input_ids.T.astype(jnp.int32)
    idxa = ids_t[:, : B // 2].reshape(-1)
    idxb = ids_t[:, B // 2:].reshape(-1)
    g2 = jnp.concatenate([gamma, gamma])
    b2 = jnp.concatenate([beta, beta])
    tp = _pack(table.T)
    hc = B_CHUNK // 2
    rows = [
        _sc_gather(
            lax.slice(idxa, (c * hc,), ((c + 1) * hc,)).reshape(hc // CH, CH),
            lax.slice(idxb, (c * hc,), ((c + 1) * hc,)).reshape(hc // CH, CH),
            tp,
        )
        for c in range(CHUNKS)
    ]
    rows2 = [r.reshape(B_CHUNK // 2, 128) for r in rows]
    out3 = _ln_chunks[0](rows2[0], g2, b2)
    for c in range(1, CHUNKS):
        out3 = _ln_chunks[c](rows2[c], g2, b2, out3)
    return out3.transpose(2, 0, 1)
